# baseline (device time: 32703 ns/iter reference)
import jax
import jax.numpy as jnp
from jax import lax
from jax.experimental import pallas as pl
from jax.experimental.pallas import tpu as pltpu

N_DEV = 8
M_BLK = 512
K_BLK = 512
N_OUT = 2048
FP8 = jnp.float8_e5m2
W_DEPTH = 4


def kernel(x, w_mat, scale_x, scale_w):
    m_total, k_shard = x.shape
    assert m_total == N_DEV * M_BLK and k_shard == K_BLK, x.shape

    def body(x_ref, w_ref, sx_ref, sw_ref, out_ref,
             xf, xq, buf, wbuf, wq, vout,
             send_sems, recv_sems, x_sems, w_sems, out_sem):
        me = lax.axis_index("i")

        barrier = pltpu.get_barrier_semaphore()
        for s in range(1, N_DEV):
            peer = lax.rem(me + s, N_DEV)
            pl.semaphore_signal(
                barrier, inc=1,
                device_id=(peer,), device_id_type=pl.DeviceIdType.MESH,
            )

        def k_block(t):
            return me if t == 0 else lax.rem(me - t + N_DEV, N_DEV)

        def w_copy(t):
            j = k_block(t)
            return pltpu.make_async_copy(
                w_ref.at[pl.ds(j * K_BLK, K_BLK), :],
                wbuf.at[t % W_DEPTH], w_sems.at[t % W_DEPTH])

        def x_stage(p, slot):
            return pltpu.make_async_copy(
                x_ref.at[pl.ds(p * M_BLK, M_BLK), :],
                xf.at[slot], x_sems.at[slot])

        for t in range(W_DEPTH):
            w_copy(t).start()

        stage_order = [lax.rem(me + s, N_DEV) for s in range(1, N_DEV)] + [me]
        x_stage(stage_order[0], 0).start()
        x_stage(stage_order[1], 1).start()
        sends = []
        for idx, p in enumerate(stage_order):
            slot = idx % 2
            x_stage(p, slot).wait()
            xq[p] = xf[slot].astype(FP8)
            if idx + 2 < N_DEV:
                x_stage(stage_order[idx + 2], slot).start()
            if idx == 0:
                pl.semaphore_wait(barrier, N_DEV - 1)
            if idx < N_DEV - 1:
                s = idx + 1
                rdma = pltpu.make_async_remote_copy(
                    src_ref=xq.at[p],
                    dst_ref=buf.at[me],
                    send_sem=send_sems.at[s],
                    recv_sem=recv_sems.at[s],
                    device_id=(p,),
                    device_id_type=pl.DeviceIdType.MESH,
                )
                rdma.start()
                sends.append(rdma)

        for t in range(N_DEV):
            w_copy(t).wait()
            j = k_block(t)
            wq[j] = wbuf[t % W_DEPTH].astype(FP8)
            if t + W_DEPTH < N_DEV:
                w_copy(t + W_DEPTH).start()

        def mm(a, j):
            return lax.dot_general(
                a, wq[j], (((1,), (0,)), ((), ())),
                preferred_element_type=jnp.float32,
            )

        acc = mm(xq[me], me)
        for t in range(1, N_DEV):
            src_dev = k_block(t)
            recv = pltpu.make_async_remote_copy(
                src_ref=xq.at[0],
                dst_ref=buf.at[src_dev],
                send_sem=send_sems.at[t],
                recv_sem=recv_sems.at[t],
                device_id=(me,),
                device_id_type=pl.DeviceIdType.MESH,
            )
            recv.wait_recv()
            acc += mm(buf[src_dev], src_dev)

        vout[:, :] = jnp.maximum(acc * (sx_ref[0] * sw_ref[0]), 0.0)
        out_cp = pltpu.make_async_copy(vout, out_ref, out_sem)
        out_cp.start()
        out_cp.wait()

        for rdma in sends:
            rdma.wait_send()

    return pl.pallas_call(
        body,
        out_shape=jax.ShapeDtypeStruct((M_BLK, N_OUT), jnp.float32),
        in_specs=[
            pl.BlockSpec(memory_space=pl.ANY),
            pl.BlockSpec(memory_space=pl.ANY),
            pl.BlockSpec(memory_space=pltpu.SMEM),
            pl.BlockSpec(memory_space=pltpu.SMEM),
        ],
        out_specs=pl.BlockSpec(memory_space=pl.ANY),
        scratch_shapes=[
            pltpu.VMEM((2, M_BLK, K_BLK), jnp.float32),
            pltpu.VMEM((N_DEV, M_BLK, K_BLK), FP8),
            pltpu.VMEM((N_DEV, M_BLK, K_BLK), FP8),
            pltpu.VMEM((W_DEPTH, K_BLK, N_OUT), jnp.float32),
            pltpu.VMEM((N_DEV, K_BLK, N_OUT), FP8),
            pltpu.VMEM((M_BLK, N_OUT), jnp.float32),
            pltpu.SemaphoreType.DMA((N_DEV,)),
            pltpu.SemaphoreType.DMA((N_DEV,)),
            pltpu.SemaphoreType.DMA((2,)),
            pltpu.SemaphoreType.DMA((W_DEPTH,)),
            pltpu.SemaphoreType.DMA(()),
        ],
        compiler_params=pltpu.CompilerParams(
            collective_id=0,
            vmem_limit_bytes=63 * 1024 * 1024,
        ),
    )(x, w_mat, scale_x, scale_w)


# device time: 31534 ns/iter; 1.0371x vs baseline; 1.0371x over previous
import jax
import jax.numpy as jnp
from jax import lax
from jax.experimental import pallas as pl
from jax.experimental.pallas import tpu as pltpu

N_DEV = 8
M_BLK = 512
K_BLK = 512
N_OUT = 2048
FP8 = jnp.float8_e5m2
W_DEPTH = 2


def kernel(x, w_mat, scale_x, scale_w):
    m_total, k_shard = x.shape
    assert m_total == N_DEV * M_BLK and k_shard == K_BLK, x.shape

    def body(x_ref, w_ref, sx_ref, sw_ref, out_ref,
             xf, xq, buf, wbuf, wq, vout,
             send_sems, recv_sems, x_sems, w_sems, out_sem):
        me = lax.axis_index("i")

        barrier = pltpu.get_barrier_semaphore()
        for s in range(1, N_DEV):
            peer = lax.rem(me + s, N_DEV)
            pl.semaphore_signal(
                barrier, inc=1,
                device_id=(peer,), device_id_type=pl.DeviceIdType.MESH,
            )

        def k_block(t):
            return me if t == 0 else lax.rem(me - t + N_DEV, N_DEV)

        def w_copy(t):
            j = k_block(t)
            return pltpu.make_async_copy(
                w_ref.at[pl.ds(j * K_BLK, K_BLK), :],
                wbuf.at[t % W_DEPTH], w_sems.at[t % W_DEPTH])

        def x_stage(p, slot):
            return pltpu.make_async_copy(
                x_ref.at[pl.ds(p * M_BLK, M_BLK), :],
                xf.at[slot], x_sems.at[slot])

        stage_order = [lax.rem(me + s, N_DEV) for s in range(1, N_DEV)] + [me]
        x_stage(stage_order[0], 0).start()
        x_stage(stage_order[1], 1).start()
        for t in range(W_DEPTH):
            w_copy(t).start()
        sends = []
        for idx, p in enumerate(stage_order):
            slot = idx % 2
            x_stage(p, slot).wait()
            xq[p] = xf[slot].astype(FP8)
            if idx + 2 < N_DEV:
                x_stage(stage_order[idx + 2], slot).start()
            if idx == 0:
                pl.semaphore_wait(barrier, N_DEV - 1)
            if idx < N_DEV - 1:
                s = idx + 1
                rdma = pltpu.make_async_remote_copy(
                    src_ref=xq.at[p],
                    dst_ref=buf.at[me],
                    send_sem=send_sems.at[s],
                    recv_sem=recv_sems.at[s],
                    device_id=(p,),
                    device_id_type=pl.DeviceIdType.MESH,
                )
                rdma.start()
                sends.append(rdma)

        for t in range(N_DEV):
            w_copy(t).wait()
            j = k_block(t)
            wq[j] = wbuf[t % W_DEPTH].astype(FP8)
            if t + W_DEPTH < N_DEV:
                w_copy(t + W_DEPTH).start()

        def mm(a, j):
            return lax.dot_general(
                a, wq[j], (((1,), (0,)), ((), ())),
                preferred_element_type=jnp.float32,
            )

        acc = mm(xq[me], me)
        for t in range(1, N_DEV):
            src_dev = k_block(t)
            recv = pltpu.make_async_remote_copy(
                src_ref=xq.at[0],
                dst_ref=buf.at[src_dev],
                send_sem=send_sems.at[t],
                recv_sem=recv_sems.at[t],
                device_id=(me,),
                device_id_type=pl.DeviceIdType.MESH,
            )
            recv.wait_recv()
            acc += mm(buf[src_dev], src_dev)

        vout[:, :] = jnp.maximum(acc * (sx_ref[0] * sw_ref[0]), 0.0)
        out_cp = pltpu.make_async_copy(vout, out_ref, out_sem)
        out_cp.start()
        out_cp.wait()

        for rdma in sends:
            rdma.wait_send()

    return pl.pallas_call(
        body,
        out_shape=jax.ShapeDtypeStruct((M_BLK, N_OUT), jnp.float32),
        in_specs=[
            pl.BlockSpec(memory_space=pl.ANY),
            pl.BlockSpec(memory_space=pl.ANY),
            pl.BlockSpec(memory_space=pltpu.SMEM),
            pl.BlockSpec(memory_space=pltpu.SMEM),
        ],
        out_specs=pl.BlockSpec(memory_space=pl.ANY),
        scratch_shapes=[
            pltpu.VMEM((2, M_BLK, K_BLK), jnp.float32),
            pltpu.VMEM((N_DEV, M_BLK, K_BLK), FP8),
            pltpu.VMEM((N_DEV, M_BLK, K_BLK), FP8),
            pltpu.VMEM((W_DEPTH, K_BLK, N_OUT), jnp.float32),
            pltpu.VMEM((N_DEV, K_BLK, N_OUT), FP8),
            pltpu.VMEM((M_BLK, N_OUT), jnp.float32),
            pltpu.SemaphoreType.DMA((N_DEV,)),
            pltpu.SemaphoreType.DMA((N_DEV,)),
            pltpu.SemaphoreType.DMA((2,)),
            pltpu.SemaphoreType.DMA((W_DEPTH,)),
            pltpu.SemaphoreType.DMA(()),
        ],
        compiler_params=pltpu.CompilerParams(
            collective_id=0,
            vmem_limit_bytes=63 * 1024 * 1024,
        ),
    )(x, w_mat, scale_x, scale_w)
